# baseline (speedup 1.0000x reference)
import functools

import jax
import jax.numpy as jnp
from jax import lax
from jax.experimental import pallas as pl
from jax.experimental.pallas import tpu as pltpu
from jax.experimental.pallas import tpu_sc as plsc

_VOCAB = 1000000
_NUM_LABELS = 50
_WORD_DIM = 50
_POS_DIM = 15


def _sc_gather(table, idx, B, D):
    info = plsc.get_sparse_core_info()
    nw = info.num_cores * info.num_subcores
    b_per_w = B // nw
    mesh = plsc.VectorSubcoreMesh(core_axis_name="c", subcore_axis_name="s")

    @functools.partial(
        pl.kernel,
        mesh=mesh,
        out_type=jax.ShapeDtypeStruct((B, D), jnp.float32),
        scratch_types=[
            pltpu.VMEM((b_per_w,), jnp.int32),
            pltpu.SemaphoreType.DMA,
        ],
    )
    def gather_k(table_hbm, idx_hbm, out_hbm, idx_v, sem):
        wid = lax.axis_index("s") * info.num_cores + lax.axis_index("c")
        base = wid * b_per_w
        pltpu.sync_copy(idx_hbm.at[pl.ds(base, b_per_w)], idx_v)

        def body(g, carry):
            vec = idx_v[pl.ds(g * 16, 16)]
            for j in range(16):
                r = vec[j]
                pltpu.async_copy(
                    table_hbm.at[pl.ds(r, 1)],
                    out_hbm.at[pl.ds(base + g * 16 + j, 1)],
                    sem,
                )
            return carry

        lax.fori_loop(0, b_per_w // 16, body, 0)
        pltpu.make_async_copy(
            table_hbm.at[pl.ds(0, b_per_w)],
            out_hbm.at[pl.ds(base, b_per_w)],
            sem,
        ).wait()

    return gather_k(table, idx)


def _tc_body(emb_ref, pos_ref, W_ref, ptab_ref, b_ref, out_ref):
    x = emb_ref[...]
    W = W_ref[...]
    Ww = W[:, :_WORD_DIM]
    Wp = W[:, _WORD_DIM:]
    P = lax.dot_general(ptab_ref[...], Wp, (((1,), (1,)), ((), ())),
                        precision=lax.Precision.HIGHEST)
    labels = lax.broadcasted_iota(jnp.int32, (1, _NUM_LABELS), 1)
    onehot = (pos_ref[...] == labels).astype(jnp.float32)
    scores = lax.dot_general(x, Ww, (((1,), (1,)), ((), ())),
                             precision=lax.Precision.HIGHEST)
    scores = scores + lax.dot_general(onehot, P, (((1,), (0,)), ((), ())),
                                      precision=lax.Precision.HIGHEST)
    out_ref[...] = scores + b_ref[...]


def kernel(word_ids, prev_pos, word_table, pos_table, W, b):
    B = word_ids.shape[0]
    emb = _sc_gather(word_table, word_ids.astype(jnp.int32), B, _WORD_DIM)

    blk = 2048
    grid = (B // blk,)
    scores = pl.pallas_call(
        _tc_body,
        grid=grid,
        in_specs=[
            pl.BlockSpec((blk, _WORD_DIM), lambda i: (i, 0)),
            pl.BlockSpec((blk, 1), lambda i: (i, 0)),
            pl.BlockSpec((_NUM_LABELS, _WORD_DIM + _POS_DIM), lambda i: (0, 0)),
            pl.BlockSpec((_NUM_LABELS, _POS_DIM), lambda i: (0, 0)),
            pl.BlockSpec((1, _NUM_LABELS), lambda i: (0, 0)),
        ],
        out_specs=pl.BlockSpec((blk, _NUM_LABELS), lambda i: (i, 0)),
        out_shape=jax.ShapeDtypeStruct((B, _NUM_LABELS), jnp.float32),
    )(emb, prev_pos.astype(jnp.int32).reshape(B, 1), W, pos_table,
      b.reshape(1, _NUM_LABELS))
    return scores


# speedup vs baseline: 1.6413x; 1.6413x over previous
import functools

import jax
import jax.numpy as jnp
from jax import lax
from jax.experimental import pallas as pl
from jax.experimental.pallas import tpu as pltpu
from jax.experimental.pallas import tpu_sc as plsc

_VOCAB = 1000000
_NUM_LABELS = 50
_WORD_DIM = 50
_POS_DIM = 15


def _sc_gather(table, idx, B, D):
    info = plsc.get_sparse_core_info()
    nw = info.num_cores * info.num_subcores
    b_per_w = B // nw
    mesh = plsc.VectorSubcoreMesh(core_axis_name="c", subcore_axis_name="s")

    @functools.partial(
        pl.kernel,
        mesh=mesh,
        out_type=jax.ShapeDtypeStruct((B, D), jnp.float32),
        scratch_types=[
            pltpu.VMEM((b_per_w,), jnp.int32),
            pltpu.VMEM((b_per_w, D), jnp.float32),
            pltpu.SemaphoreType.DMA,
        ],
    )
    def gather_k(table_hbm, idx_hbm, out_hbm, idx_v, rows_v, sem):
        wid = lax.axis_index("s") * info.num_cores + lax.axis_index("c")
        base = wid * b_per_w
        pltpu.sync_copy(idx_hbm.at[pl.ds(base, b_per_w)], idx_v)

        def body(g, carry):
            vec = idx_v[pl.ds(g * 16, 16)]
            for j in range(16):
                r = vec[j]
                pltpu.async_copy(
                    table_hbm.at[pl.ds(r, 1)],
                    rows_v.at[pl.ds(g * 16 + j, 1)],
                    sem,
                )
            return carry

        lax.fori_loop(0, b_per_w // 16, body, 0)
        pltpu.make_async_copy(
            table_hbm.at[pl.ds(0, b_per_w)], rows_v, sem
        ).wait()
        pltpu.sync_copy(rows_v, out_hbm.at[pl.ds(base, b_per_w)])

    return gather_k(table, idx)


def _tc_body(emb_ref, pos_ref, W_ref, ptab_ref, b_ref, out_ref):
    x = emb_ref[...]
    W = W_ref[...]
    Ww = W[:, :_WORD_DIM]
    Wp = W[:, _WORD_DIM:]
    P = lax.dot_general(ptab_ref[...], Wp, (((1,), (1,)), ((), ())),
                        precision=lax.Precision.HIGHEST)
    labels = lax.broadcasted_iota(jnp.int32, (1, _NUM_LABELS), 1)
    onehot = (pos_ref[...] == labels).astype(jnp.float32)
    scores = lax.dot_general(x, Ww, (((1,), (1,)), ((), ())),
                             precision=lax.Precision.HIGHEST)
    scores = scores + lax.dot_general(onehot, P, (((1,), (0,)), ((), ())),
                                      precision=lax.Precision.HIGHEST)
    out_ref[...] = scores + b_ref[...]


def kernel(word_ids, prev_pos, word_table, pos_table, W, b):
    B = word_ids.shape[0]
    emb = _sc_gather(word_table, word_ids.astype(jnp.int32), B, _WORD_DIM)

    blk = 2048
    grid = (B // blk,)
    scores = pl.pallas_call(
        _tc_body,
        grid=grid,
        in_specs=[
            pl.BlockSpec((blk, _WORD_DIM), lambda i: (i, 0)),
            pl.BlockSpec((blk, 1), lambda i: (i, 0)),
            pl.BlockSpec((_NUM_LABELS, _WORD_DIM + _POS_DIM), lambda i: (0, 0)),
            pl.BlockSpec((_NUM_LABELS, _POS_DIM), lambda i: (0, 0)),
            pl.BlockSpec((1, _NUM_LABELS), lambda i: (0, 0)),
        ],
        out_specs=pl.BlockSpec((blk, _NUM_LABELS), lambda i: (i, 0)),
        out_shape=jax.ShapeDtypeStruct((B, _NUM_LABELS), jnp.float32),
    )(emb, prev_pos.astype(jnp.int32).reshape(B, 1), W, pos_table,
      b.reshape(1, _NUM_LABELS))
    return scores
